# baseline (speedup 1.0000x reference)
import jax
import jax.numpy as jnp
from jax import lax
from jax.experimental import pallas as pl
from jax.experimental.pallas import tpu as pltpu
from jax.experimental.pallas import tpu_sc as plsc

_D = 64
_NC, _NS = 2, 16
_NW = _NC * _NS
_LANE = 128
_GPB = 5
_NBUF = 2


def _emb_body(idx_hbm, table_hbm, out_hbm, idx_v, rows_v, gsem, osem):
    wid = lax.axis_index("s") * _NC + lax.axis_index("c")
    n_idxrow = idx_hbm.shape[1]
    n_block = n_idxrow // _GPB
    base = wid * n_idxrow
    pltpu.sync_copy(idx_hbm.at[wid], idx_v)

    def start_gathers(blk, B):
        for k in range(_GPB):
            pltpu.async_copy(
                table_hbm.at[idx_v.at[blk * _GPB + k]],
                rows_v.at[B].at[k],
                gsem.at[B],
            )

    def wait_gathers(blk, B):
        pltpu.make_async_copy(
            out_hbm.at[pl.ds(base + blk * _GPB, _GPB)], rows_v.at[B],
            gsem.at[B]).wait()

    def out_slice(blk):
        return out_hbm.at[pl.ds(base + blk * _GPB, _GPB)]

    def start_out(blk, B):
        pltpu.async_copy(rows_v.at[B], out_slice(blk), osem.at[B])

    def wait_out(blk, B):
        pltpu.make_async_copy(rows_v.at[B], out_slice(blk), osem.at[B]).wait()

    for B in range(_NBUF):
        start_gathers(B, B)

    n_group = n_block // _NBUF

    def body(g, carry):
        for B in range(_NBUF):
            blk = g * _NBUF + B
            wait_gathers(blk, B)
            start_out(blk, B)
            wait_out(blk, B)
            start_gathers(blk + _NBUF, B)
        return carry

    lax.fori_loop(0, n_group - 1, body, 0)

    for B in range(_NBUF):
        blk = (n_group - 1) * _NBUF + B
        wait_gathers(blk, B)
        start_out(blk, B)
    for B in range(_NBUF):
        blk = (n_group - 1) * _NBUF + B
        wait_out(blk, B)


def kernel(token_ids, weight):
    b, s = token_ids.shape
    total = b * s
    n_idxrow = total // (_NW * _LANE)
    idx = token_ids.reshape(_NW, n_idxrow, _LANE).astype(jnp.int32)
    mesh = plsc.VectorSubcoreMesh(core_axis_name="c", subcore_axis_name="s")
    out = pl.kernel(
        _emb_body,
        out_type=jax.ShapeDtypeStruct((total // _LANE, _LANE, _D), jnp.float32),
        mesh=mesh,
        scratch_types=[
            pltpu.VMEM((n_idxrow, _LANE), jnp.int32),
            pltpu.VMEM((_NBUF, _GPB, _LANE, _D), jnp.float32),
            pltpu.SemaphoreType.DMA((_NBUF,)),
            pltpu.SemaphoreType.DMA((_NBUF,)),
        ],
        compiler_params=pltpu.CompilerParams(use_tc_tiling_on_sc=False),
    )(idx, weight)
    return out.reshape(b, s, _D)
